# baseline (speedup 1.0000x reference)
import jax
import jax.numpy as jnp
from jax.experimental import pallas as pl

_SPLITS = {0: 4, 1: 2, 2: 1}


def _heads_body(*refs):
    n0, n1, n2 = _SPLITS[0], _SPLITS[1], _SPLITS[2]
    xs0 = refs[:n0]
    xs1 = refs[n0:n0 + n1]
    xs2 = refs[n0 + n1:n0 + n1 + n2]
    w0, b0, w1, b1, w2, b2 = refs[n0 + n1 + n2:n0 + n1 + n2 + 6]
    o0, o1, o2 = refs[-3:]

    def head(xs, w, b, o):
        ck = w.shape[1] // len(xs)
        acc = b[...]
        for j, x in enumerate(xs):
            acc = acc + jnp.dot(w[:, j * ck:(j + 1) * ck], x[0],
                                preferred_element_type=jnp.float32)
        o[0] = acc

    head(xs0, w0, b0, o0)
    head(xs1, w1, b1, o1)
    head(xs2, w2, b2, o2)


def kernel(feats_0, feats_1, feats_2, W0, b0, W1, b1, W2, b2):
    B = feats_0.shape[0]
    shapes = [feats_0.shape, feats_1.shape, feats_2.shape]
    xs = [f.reshape(f.shape[0], f.shape[1], f.shape[2] * f.shape[3])
          for f in (feats_0, feats_1, feats_2)]
    ws = [W0, W1, W2]
    bs = [b.reshape(-1, 1) for b in (b0, b1, b2)]
    out_dim = W0.shape[0]

    def full_spec(a):
        return pl.BlockSpec(a.shape, lambda b: (0,) * a.ndim)

    operands = []
    in_specs = []
    for i, x in enumerate(xs):
        n = _SPLITS[i]
        ck = x.shape[1] // n
        for j in range(n):
            operands.append(x)
            in_specs.append(
                pl.BlockSpec((1, ck, x.shape[2]),
                             lambda b, jj=j: (b, jj, 0)))
    for w, bia in zip(ws, bs):
        operands.extend([w, bia])
        in_specs.extend([full_spec(w), full_spec(bia)])

    out_shapes = [jax.ShapeDtypeStruct((B, out_dim, x.shape[2]), jnp.float32)
                  for x in xs]
    out_specs = [pl.BlockSpec((1, out_dim, x.shape[2]), lambda b: (b, 0, 0))
                 for x in xs]

    outs = pl.pallas_call(
        _heads_body,
        grid=(B,),
        in_specs=in_specs,
        out_specs=out_specs,
        out_shape=out_shapes,
    )(*operands)

    return tuple(
        o.reshape(s[0], out_dim, s[2], s[3]) for o, s in zip(outs, shapes)
    )


# speedup vs baseline: 1.0120x; 1.0120x over previous
import jax
import jax.numpy as jnp
from jax.experimental import pallas as pl
from jax.experimental.pallas import tpu as pltpu

_DEPTH = 4


def _heads_body(x0, x1, x2, w0, b0, w1, b1, w2, b2,
                o0, o1, o2,
                ib0, ib1, ib2, ob0, ob1, ob2, isem, osem):
    xs = (x0, x1, x2)
    os_ = (o0, o1, o2)
    ibs = (ib0, ib1, ib2)
    obs = (ob0, ob1, ob2)
    ws = (w0, w1, w2)
    bs = (b0, b1, b2)
    B = x0.shape[0]
    D = _DEPTH

    def in_copy(s, b):
        return pltpu.make_async_copy(
            xs[s].at[b], ibs[s].at[b % D], isem.at[s, b % D])

    def out_copy(s, b):
        return pltpu.make_async_copy(
            obs[s].at[b % D], os_[s].at[b], osem.at[s, b % D])

    for b in range(D):
        for s in range(3):
            in_copy(s, b).start()

    for b in range(B):
        for s in range(3):
            in_copy(s, b).wait()
            if b >= D:
                out_copy(s, b - D).wait()
            obs[s][b % D] = (
                jnp.dot(ws[s][...], ibs[s][b % D],
                        preferred_element_type=jnp.float32) + bs[s][...])
            out_copy(s, b).start()
            if b + D < B:
                in_copy(s, b + D).start()

    for b in range(max(0, B - D), B):
        for s in range(3):
            out_copy(s, b).wait()


def kernel(feats_0, feats_1, feats_2, W0, b0, W1, b1, W2, b2):
    B = feats_0.shape[0]
    shapes = [feats_0.shape, feats_1.shape, feats_2.shape]
    xs = [f.reshape(f.shape[0], f.shape[1], f.shape[2] * f.shape[3])
          for f in (feats_0, feats_1, feats_2)]
    ws = [W0, W1, W2]
    bs = [b.reshape(-1, 1) for b in (b0, b1, b2)]
    out_dim = W0.shape[0]

    hbm = pl.BlockSpec(memory_space=pltpu.MemorySpace.HBM)
    vmem = pl.BlockSpec(memory_space=pltpu.MemorySpace.VMEM)

    in_specs = [hbm, hbm, hbm] + [vmem] * 6
    out_shapes = [jax.ShapeDtypeStruct((B, out_dim, x.shape[2]), jnp.float32)
                  for x in xs]
    out_specs = [hbm, hbm, hbm]

    scratch = (
        [pltpu.VMEM((_DEPTH, x.shape[1], x.shape[2]), jnp.float32) for x in xs]
        + [pltpu.VMEM((_DEPTH, out_dim, x.shape[2]), jnp.float32) for x in xs]
        + [pltpu.SemaphoreType.DMA((3, _DEPTH)),
           pltpu.SemaphoreType.DMA((3, _DEPTH))]
    )

    outs = pl.pallas_call(
        _heads_body,
        in_specs=in_specs,
        out_specs=out_specs,
        out_shape=out_shapes,
        scratch_shapes=scratch,
        compiler_params=pltpu.CompilerParams(
            vmem_limit_bytes=100 * 1024 * 1024),
    )(xs[0], xs[1], xs[2], ws[0], bs[0], ws[1], bs[1], ws[2], bs[2])

    return tuple(
        o.reshape(s[0], out_dim, s[2], s[3]) for o, s in zip(outs, shapes)
    )


# speedup vs baseline: 3.5005x; 3.4588x over previous
import jax
import jax.numpy as jnp
from jax.experimental import pallas as pl


def _heads_body(x0, w0, b0, x1, w1, b1, x2, w2, b2, o0, o1, o2):
    dn = (((1,), (1,)), ((), ()))
    o0[0] = jax.lax.dot_general(
        x0[0], w0[...], dn, preferred_element_type=jnp.float32) + b0[...]
    o1[0] = jax.lax.dot_general(
        x1[0], w1[...], dn, preferred_element_type=jnp.float32) + b1[...]
    o2[0] = jax.lax.dot_general(
        x2[0], w2[...], dn, preferred_element_type=jnp.float32) + b2[...]


def kernel(feats_0, feats_1, feats_2, W0, b0, W1, b1, W2, b2):
    B = feats_0.shape[0]
    shapes = [feats_0.shape, feats_1.shape, feats_2.shape]
    xs = [jnp.transpose(f, (0, 2, 3, 1)).reshape(
              f.shape[0], f.shape[2] * f.shape[3], f.shape[1])
          for f in (feats_0, feats_1, feats_2)]
    ws = [W0, W1, W2]
    bs = [b.reshape(1, -1) for b in (b0, b1, b2)]
    out_dim = W0.shape[0]

    def feat_spec(x):
        return pl.BlockSpec((1, x.shape[1], x.shape[2]), lambda b: (b, 0, 0))

    def full_spec(a):
        return pl.BlockSpec(a.shape, lambda b: (0,) * a.ndim)

    in_specs = []
    operands = []
    for x, w, bia in zip(xs, ws, bs):
        operands.extend([x, w, bia])
        in_specs.extend([feat_spec(x), full_spec(w), full_spec(bia)])

    out_shapes = [jax.ShapeDtypeStruct((B, x.shape[1], out_dim), jnp.float32)
                  for x in xs]
    out_specs = [pl.BlockSpec((1, x.shape[1], out_dim), lambda b: (b, 0, 0))
                 for x in xs]

    outs = pl.pallas_call(
        _heads_body,
        grid=(B,),
        in_specs=in_specs,
        out_specs=out_specs,
        out_shape=out_shapes,
    )(*operands)

    return tuple(
        jnp.transpose(o.reshape(s[0], s[2], s[3], out_dim), (0, 3, 1, 2))
        for o, s in zip(outs, shapes)
    )
